# Initial kernel scaffold; baseline (speedup 1.0000x reference)
import jax
import jax.numpy as jnp
from jax.experimental import pallas as pl


def kernel(x, embed_table, W):
    raise NotImplementedError("write your pallas kernel here")


# speedup vs baseline: 1.3547x; 1.3547x over previous
import functools

import jax
import jax.numpy as jnp
from jax import lax
from jax.experimental import pallas as pl
from jax.experimental.pallas import tpu as pltpu
from jax.experimental.pallas import tpu_sc as plsc

D_LOW = 128
D_HIGH = 1024
IDX_CHUNK = 128


@functools.lru_cache(maxsize=None)
def _sc_gather_fn(n_tokens, vocab):
    info = plsc.get_sparse_core_info()
    nw = info.num_cores * info.num_subcores
    b_per_w = n_tokens // nw
    n_chunks = b_per_w // IDX_CHUNK
    mesh = plsc.VectorSubcoreMesh(core_axis_name="c", subcore_axis_name="s")

    @functools.partial(
        pl.kernel,
        mesh=mesh,
        out_type=jax.ShapeDtypeStruct((n_tokens, D_LOW), jnp.float32),
        scratch_types=[
            pltpu.VMEM((n_chunks, IDX_CHUNK), jnp.int32),
            pltpu.VMEM((b_per_w, D_LOW), jnp.float32),
            pltpu.SemaphoreType.DMA,
        ],
    )
    def gather(table_hbm, idx_hbm, out_hbm, idx_v, rows_v, sem):
        wid = lax.axis_index("s") * info.num_cores + lax.axis_index("c")
        base = wid * b_per_w
        pltpu.sync_copy(idx_hbm.at[pl.ds(wid * n_chunks, n_chunks)], idx_v)
        copies = []
        for j in range(n_chunks):
            copies.append(
                pltpu.async_copy(
                    table_hbm.at[idx_v.at[j]],
                    rows_v.at[pl.ds(j * IDX_CHUNK, IDX_CHUNK)],
                    sem,
                )
            )
        for c in copies:
            c.wait()
        pltpu.sync_copy(rows_v, out_hbm.at[pl.ds(base, b_per_w)])

    return gather


def _tc_project(low, w):
    n = low.shape[0]
    blk = 1024

    def body(low_ref, w_ref, out_ref):
        out_ref[...] = jnp.dot(
            low_ref[...], w_ref[...], preferred_element_type=jnp.float32
        )

    return pl.pallas_call(
        body,
        grid=(n // blk,),
        in_specs=[
            pl.BlockSpec((blk, D_LOW), lambda i: (i, 0)),
            pl.BlockSpec((D_LOW, D_HIGH), lambda i: (0, 0)),
        ],
        out_specs=pl.BlockSpec((blk, D_HIGH), lambda i: (i, 0)),
        out_shape=jax.ShapeDtypeStruct((n, D_HIGH), jnp.float32),
    )(low, w)


def kernel(x, embed_table, W):
    b, s = x.shape
    n = b * s
    idx = x.reshape(n // IDX_CHUNK, IDX_CHUNK).astype(jnp.int32)
    low = _sc_gather_fn(n, embed_table.shape[0])(embed_table, idx)
    out = _tc_project(low, W)
    return out.reshape(b, s, D_HIGH)
